# baseline (speedup 1.0000x reference)
import functools

import jax
import jax.numpy as jnp
from jax import lax
from jax.experimental import pallas as pl
from jax.experimental.pallas import tpu as pltpu
from jax.experimental.pallas import tpu_sc as plsc

BATCH = 4096
SEQ_LEN = 200
EMBED_DIM = 64

TOT = BATCH * SEQ_LEN
IDX_MINOR = 128
IDX_ROWS = TOT // IDX_MINOR

NUM_WORKERS = 32
ROWS_PER_W = IDX_ROWS // NUM_WORKERS
NB = 8
CHUNK = NB * IDX_MINOR
G = ROWS_PER_W // NB

_mesh = plsc.VectorSubcoreMesh(core_axis_name="c", subcore_axis_name="s")


@functools.partial(
    pl.kernel,
    mesh=_mesh,
    out_type=jax.ShapeDtypeStruct((IDX_ROWS, IDX_MINOR, EMBED_DIM), jnp.float32),
    scratch_types=[
        pltpu.VMEM((NB, IDX_MINOR), jnp.int32),
        pltpu.VMEM((NB, IDX_MINOR, EMBED_DIM), jnp.float32),
        pltpu.SemaphoreType.DMA,
    ],
    compiler_params=pltpu.CompilerParams(use_tc_tiling_on_sc=False),
)
def _sc_gather(idx_hbm, table_hbm, out_hbm, idx_v, rows_v, sem):
    wid = lax.axis_index("s") * 2 + lax.axis_index("c")
    row0 = wid * ROWS_PER_W

    def body(g, _):
        r = row0 + g * NB
        pltpu.sync_copy(idx_hbm.at[pl.ds(r, NB), :], idx_v)
        handles = [
            pltpu.async_copy(table_hbm.at[idx_v.at[j]], rows_v.at[j], sem)
            for j in range(NB)
        ]
        for h in handles:
            h.wait()
        pltpu.sync_copy(rows_v, out_hbm.at[pl.ds(r, NB)])
        return 0

    lax.fori_loop(0, G, body, 0)


def _mask_body(idx_ref, mask_ref):
    mask_ref[...] = (idx_ref[...] != 0).astype(jnp.int32)


_mask_call = pl.pallas_call(
    _mask_body,
    out_shape=jax.ShapeDtypeStruct((BATCH, SEQ_LEN), jnp.int32),
)


def kernel(indices, table):
    idx2d = indices.reshape(IDX_ROWS, IDX_MINOR)
    rows = _sc_gather(idx2d, table)
    outputs = rows.reshape(BATCH, SEQ_LEN, EMBED_DIM)
    mask = _mask_call(indices)
    return outputs, mask


# speedup vs baseline: 1.0171x; 1.0171x over previous
import functools

import jax
import jax.numpy as jnp
from jax import lax
from jax.experimental import pallas as pl
from jax.experimental.pallas import tpu as pltpu
from jax.experimental.pallas import tpu_sc as plsc

BATCH = 4096
SEQ_LEN = 200
EMBED_DIM = 64

TOT = BATCH * SEQ_LEN
IDX_MINOR = 128
IDX_ROWS = TOT // IDX_MINOR

NUM_WORKERS = 32
ROWS_PER_W = IDX_ROWS // NUM_WORKERS
NB = 5
G = ROWS_PER_W // NB
NITER = G // 2

_mesh = plsc.VectorSubcoreMesh(core_axis_name="c", subcore_axis_name="s")


@functools.partial(
    pl.kernel,
    mesh=_mesh,
    out_type=jax.ShapeDtypeStruct((IDX_ROWS, IDX_MINOR, EMBED_DIM), jnp.float32),
    scratch_types=[
        pltpu.VMEM((2, NB, IDX_MINOR), jnp.int32),
        pltpu.VMEM((2, NB, IDX_MINOR, EMBED_DIM), jnp.float32),
        pltpu.SemaphoreType.DMA,
        pltpu.SemaphoreType.DMA,
        pltpu.SemaphoreType.DMA,
        pltpu.SemaphoreType.DMA,
    ],
    compiler_params=pltpu.CompilerParams(use_tc_tiling_on_sc=False),
)
def _sc_gather(idx_hbm, table_hbm, out_hbm, idx_v, rows_v, sg0, sg1, ss0, ss1):
    wid = lax.axis_index("s") * 2 + lax.axis_index("c")
    row0 = wid * ROWS_PER_W
    sem_g = (sg0, sg1)
    sem_st = (ss0, ss1)

    def load_idx(g, b):
        pltpu.sync_copy(idx_hbm.at[pl.ds(row0 + g * NB, NB), :], idx_v.at[b])

    def fire_gathers(b):
        for j in range(NB):
            pltpu.async_copy(
                table_hbm.at[idx_v.at[b].at[j]], rows_v.at[b].at[j], sem_g[b]
            )

    def drain_gathers(b):
        pltpu.make_async_copy(out_hbm.at[pl.ds(0, NB)], rows_v.at[b], sem_g[b]).wait()

    def store_rows(g, b):
        pltpu.async_copy(
            rows_v.at[b], out_hbm.at[pl.ds(row0 + g * NB, NB)], sem_st[b]
        )

    def drain_store(b):
        pltpu.make_async_copy(out_hbm.at[pl.ds(0, NB)], rows_v.at[b], sem_st[b]).wait()

    load_idx(0, 0)
    fire_gathers(0)

    def body(i, _):
        g0 = 2 * i

        load_idx(g0 + 1, 1)

        @pl.when(i > 0)
        def _():
            drain_store(1)

        fire_gathers(1)
        drain_gathers(0)
        store_rows(g0, 0)

        @pl.when(i < NITER - 1)
        def _():
            load_idx(g0 + 2, 0)
            drain_store(0)
            fire_gathers(0)

        drain_gathers(1)
        store_rows(g0 + 1, 1)
        return 0

    lax.fori_loop(0, NITER, body, 0)
    drain_store(0)
    drain_store(1)


def _mask_body(idx_ref, mask_ref):
    mask_ref[...] = (idx_ref[...] != 0).astype(jnp.int32)


_mask_call = pl.pallas_call(
    _mask_body,
    out_shape=jax.ShapeDtypeStruct((BATCH, SEQ_LEN), jnp.int32),
)


def kernel(indices, table):
    idx2d = indices.reshape(IDX_ROWS, IDX_MINOR)
    rows = _sc_gather(idx2d, table)
    outputs = rows.reshape(BATCH, SEQ_LEN, EMBED_DIM)
    mask = _mask_call(indices)
    return outputs, mask


# speedup vs baseline: 1.1929x; 1.1728x over previous
import functools

import jax
import jax.numpy as jnp
from jax import lax
from jax.experimental import pallas as pl
from jax.experimental.pallas import tpu as pltpu
from jax.experimental.pallas import tpu_sc as plsc

BATCH = 4096
SEQ_LEN = 200
EMBED_DIM = 64
PADDED_DIM = 128

TOT = BATCH * SEQ_LEN
IDX_MINOR = 128
IDX_ROWS = TOT // IDX_MINOR

NUM_WORKERS = 32
ROWS_PER_W = IDX_ROWS // NUM_WORKERS
NB = 8
SUB = 4
G = ROWS_PER_W // NB

_mesh = plsc.VectorSubcoreMesh(core_axis_name="c", subcore_axis_name="s")


@functools.partial(
    pl.kernel,
    mesh=_mesh,
    out_type=jax.ShapeDtypeStruct((IDX_ROWS, IDX_MINOR, PADDED_DIM), jnp.float32),
    scratch_types=[
        pltpu.VMEM((NB, IDX_MINOR), jnp.int32),
        pltpu.VMEM((SUB, IDX_MINOR, PADDED_DIM), jnp.float32),
        pltpu.SemaphoreType.DMA,
    ],
)
def _sc_gather(idx_hbm, table_hbm, out_hbm, idx_v, rows_v, sem):
    wid = lax.axis_index("s") * 2 + lax.axis_index("c")
    row0 = wid * ROWS_PER_W

    def body(g, _):
        r = row0 + g * NB
        pltpu.sync_copy(idx_hbm.at[pl.ds(r, NB), :], idx_v)
        for h in range(NB // SUB):
            handles = [
                pltpu.async_copy(
                    table_hbm.at[idx_v.at[h * SUB + j]], rows_v.at[j], sem
                )
                for j in range(SUB)
            ]
            for hd in handles:
                hd.wait()
            pltpu.sync_copy(rows_v, out_hbm.at[pl.ds(r + h * SUB, SUB)])
        return 0

    lax.fori_loop(0, G, body, 0)


def _mask_body(idx_ref, mask_ref):
    mask_ref[...] = (idx_ref[...] != 0).astype(jnp.int32)


_mask_call = pl.pallas_call(
    _mask_body,
    out_shape=jax.ShapeDtypeStruct((BATCH, SEQ_LEN), jnp.int32),
)


def kernel(indices, table):
    table_p = jnp.pad(table, ((0, 0), (0, PADDED_DIM - EMBED_DIM)))
    idx2d = indices.reshape(IDX_ROWS, IDX_MINOR)
    rows = _sc_gather(idx2d, table_p)
    outputs = rows[:, :, :EMBED_DIM].reshape(BATCH, SEQ_LEN, EMBED_DIM)
    mask = _mask_call(indices)
    return outputs, mask
